# baseline (speedup 1.0000x reference)
import functools

import jax
import jax.numpy as jnp
from jax import lax
from jax.experimental import pallas as pl
from jax.experimental.pallas import tpu as pltpu
from jax.experimental.pallas import tpu_sc as plsc

NC = 2
NS = 16
NW = NC * NS
PITCH = 129
IB = 1024


def _tc_transpose(table_t):
    v = table_t.shape[1]
    g = (v + IB - 1) // IB

    def body(x_ref, o_ref):
        x = x_ref[...]
        eye = jnp.eye(32, dtype=jnp.float32)
        for k in range(4):
            xq = x[:, 256 * k:256 * (k + 1)]
            o_ref[:, 32 * k:32 * (k + 1)] = lax.dot_general(
                xq, eye, (((0,), (0,)), ((), ())),
                preferred_element_type=jnp.float32)

    return pl.pallas_call(
        body,
        grid=(g,),
        in_specs=[pl.BlockSpec((32, IB), lambda i: (0, i))],
        out_specs=pl.BlockSpec((IB // 4, 128), lambda i: (i, 0)),
        out_shape=jax.ShapeDtypeStruct((g * IB // 4, 128), jnp.float32),
    )(table_t)


@functools.lru_cache(maxsize=None)
def _make_lookup(bsz, hist):
    tblk = bsz // 128
    tl = tblk // NW
    nu = hist * tl
    cols = tl * 128
    mesh = plsc.VectorSubcoreMesh(core_axis_name="c", subcore_axis_name="s")

    @functools.partial(
        pl.kernel,
        mesh=mesh,
        out_type=jax.ShapeDtypeStruct((hist, 4, tblk, 8, 128), jnp.float32),
        scratch_types=(
            [pltpu.VMEM((hist * cols,), jnp.int32)]
            + [pltpu.VMEM((128, 32), jnp.float32) for _ in range(2)]
            + [pltpu.VMEM((32, PITCH), jnp.float32) for _ in range(2)]
            + [pltpu.SemaphoreType.DMA for _ in range(5)]
        ),
        compiler_params=pltpu.CompilerParams(
            use_tc_tiling_on_sc=False, needs_layout_passes=False),
    )
    def k(table_hbm, idx_hbm, out_hbm, idx_all,
          rows0, rows1, outt0, outt1, isem, gsem0, gsem1, ssem0, ssem1):
        w = lax.axis_index("s") * NC + lax.axis_index("c")
        col0 = w * cols
        rows = (rows0, rows1)
        outt = (outt0, outt1)
        gsem = (gsem0, gsem1)
        ssem = (ssem0, ssem1)

        for hh in range(hist):
            pltpu.async_copy(
                idx_hbm.at[pl.ds(hh * bsz + col0, cols)],
                idx_all.at[pl.ds(hh * cols, cols)], isem)
        iot = lax.iota(jnp.int32, 16)
        for hh in range(hist):
            pltpu.make_async_copy(
                idx_hbm.at[pl.ds(0, cols)], idx_all.at[pl.ds(0, cols)],
                isem).wait()

        def remap(g, carry):
            iv = idx_all[pl.ds(g * 16, 16)]
            r = iv & (IB - 1)
            sg = (iv - r) + ((r & 255) << 2) + (r >> 8)
            idx_all[pl.ds(g * 16, 16)] = sg
            return carry

        lax.fori_loop(0, (hist * cols) // 16, remap, 0)

        def fire_gather(u, s):
            base = (u // tl) * cols + (u - (u // tl) * tl) * 128
            pltpu.async_copy(
                table_hbm.at[idx_all.at[pl.ds(base, 128)]], rows[s], gsem[s])

        def drain_write(s):
            for r in range(4):
                pltpu.make_async_copy(
                    outt[s].at[pl.ds(0, 8), pl.ds(0, 128)],
                    out_hbm.at[0, 0, 0], ssem[s]).wait()

        def proc(u, s, p):
            pltpu.make_async_copy(
                table_hbm.at[idx_all.at[pl.ds(0, 128)]], rows[s],
                gsem[s]).wait()

            @pl.when(p >= 1)
            def _():
                drain_write(s)
            for j in range(128):
                bi = iot * 0 + j
                for c0 in (0, 16):
                    v = rows[s][j, pl.ds(c0, 16)]
                    plsc.store_scatter(outt[s], [iot + c0, bi], v)
            h = u // tl
            tloc = u - h * tl
            t = (col0 // 128) + tloc
            for r in range(4):
                pltpu.async_copy(
                    outt[s].at[pl.ds(r * 8, 8), pl.ds(0, 128)],
                    out_hbm.at[h, r, t], ssem[s])

        fire_gather(0, 0)

        def body(p, carry):
            u0 = 2 * p
            fire_gather(u0 + 1, 1)
            proc(u0, 0, p)

            @pl.when(u0 + 2 < nu)
            def _():
                fire_gather(u0 + 2, 0)
            proc(u0 + 1, 1, p)
            return carry

        lax.fori_loop(0, nu // 2, body, 0)
        for s in range(2):
            drain_write(s)

    return k


def kernel(item_inputs, table):
    b, h = item_inputs.shape
    v, d = table.shape
    t4 = _tc_transpose(table.T)
    t2 = t4.reshape(t4.shape[0] * 4, 32)
    idx1 = item_inputs.T.reshape(b * h).astype(jnp.int32)
    out5 = _make_lookup(b, h)(t2, idx1)
    return out5.transpose(2, 4, 0, 1, 3).reshape(b, h, d)


# speedup vs baseline: 1.3055x; 1.3055x over previous
import functools

import jax
import jax.numpy as jnp
from jax import lax
from jax.experimental import pallas as pl
from jax.experimental.pallas import tpu as pltpu
from jax.experimental.pallas import tpu_sc as plsc

NC = 2
NS = 16
NW = NC * NS
PITCH = 129


@functools.lru_cache(maxsize=None)
def _make_lookup(bsz, hist):
    tblk = bsz // 128
    tl = tblk // NW
    nu = hist * tl
    cols = tl * 128
    mesh = plsc.VectorSubcoreMesh(core_axis_name="c", subcore_axis_name="s")

    @functools.partial(
        pl.kernel,
        mesh=mesh,
        out_type=jax.ShapeDtypeStruct((hist, 4, tblk, 8, 128), jnp.float32),
        scratch_types=(
            [pltpu.VMEM((hist * cols,), jnp.int32)]
            + [pltpu.VMEM((128, 32), jnp.float32) for _ in range(2)]
            + [pltpu.VMEM((32, PITCH), jnp.float32) for _ in range(2)]
            + [pltpu.SemaphoreType.DMA for _ in range(5)]
        ),
        compiler_params=pltpu.CompilerParams(
            use_tc_tiling_on_sc=False, needs_layout_passes=False),
    )
    def k(table_hbm, idx_hbm, out_hbm, idx_all,
          rows0, rows1, outt0, outt1, isem, gsem0, gsem1, ssem0, ssem1):
        w = lax.axis_index("s") * NC + lax.axis_index("c")
        col0 = w * cols
        rows = (rows0, rows1)
        outt = (outt0, outt1)
        gsem = (gsem0, gsem1)
        ssem = (ssem0, ssem1)

        for hh in range(hist):
            pltpu.async_copy(
                idx_hbm.at[pl.ds(hh * bsz + col0, cols)],
                idx_all.at[pl.ds(hh * cols, cols)], isem)
        iot = lax.iota(jnp.int32, 16)
        for hh in range(hist):
            pltpu.make_async_copy(
                idx_hbm.at[pl.ds(0, cols)], idx_all.at[pl.ds(0, cols)],
                isem).wait()

        def fire_gather(u, s):
            base = (u // tl) * cols + (u - (u // tl) * tl) * 128
            pltpu.async_copy(
                table_hbm.at[idx_all.at[pl.ds(base, 128)]], rows[s], gsem[s])

        def drain_write(s):
            for r in range(4):
                pltpu.make_async_copy(
                    outt[s].at[pl.ds(0, 8), pl.ds(0, 128)],
                    out_hbm.at[0, 0, 0], ssem[s]).wait()

        def proc(u, s, p):
            pltpu.make_async_copy(
                table_hbm.at[idx_all.at[pl.ds(0, 128)]], rows[s],
                gsem[s]).wait()

            @pl.when(p >= 1)
            def _():
                drain_write(s)
            for j in range(128):
                bi = iot * 0 + j
                for c0 in (0, 16):
                    v = rows[s][j, pl.ds(c0, 16)]
                    plsc.store_scatter(outt[s], [iot + c0, bi], v)
            h = u // tl
            tloc = u - h * tl
            t = (col0 // 128) + tloc
            for r in range(4):
                pltpu.async_copy(
                    outt[s].at[pl.ds(r * 8, 8), pl.ds(0, 128)],
                    out_hbm.at[h, r, t], ssem[s])

        fire_gather(0, 0)

        def body(p, carry):
            u0 = 2 * p
            fire_gather(u0 + 1, 1)
            proc(u0, 0, p)

            @pl.when(u0 + 2 < nu)
            def _():
                fire_gather(u0 + 2, 0)
            proc(u0 + 1, 1, p)
            return carry

        lax.fori_loop(0, nu // 2, body, 0)
        for s in range(2):
            drain_write(s)

    return k


def kernel(item_inputs, table):
    b, h = item_inputs.shape
    v, d = table.shape
    idx1 = item_inputs.T.reshape(b * h).astype(jnp.int32)
    out5 = _make_lookup(b, h)(table, idx1)
    return out5.transpose(2, 4, 0, 1, 3).reshape(b, h, d)
